# Initial kernel scaffold; baseline (speedup 1.0000x reference)
import jax
import jax.numpy as jnp
from jax.experimental import pallas as pl


def kernel(context, table):
    raise NotImplementedError("write your pallas kernel here")


# speedup vs baseline: 2.9360x; 2.9360x over previous
import functools

import jax
import jax.numpy as jnp
from jax import lax
from jax.experimental import pallas as pl
from jax.experimental.pallas import tpu as pltpu
from jax.experimental.pallas import tpu_sc as plsc

EMB = 128
B_ROWS = 4096
SEQ = 50
TOT = B_ROWS * SEQ
NW = 32
PER_W = TOT // NW
CHUNK = 128
NCH = PER_W // CHUNK

_mesh = plsc.VectorSubcoreMesh(core_axis_name="c", subcore_axis_name="s")


@functools.partial(
    pl.kernel,
    mesh=_mesh,
    out_type=(
        jax.ShapeDtypeStruct((TOT, EMB), jnp.float32),
        jax.ShapeDtypeStruct((TOT,), jnp.int32),
    ),
    scratch_types=[
        pltpu.VMEM((PER_W,), jnp.int32),
        pltpu.VMEM((PER_W,), jnp.int32),
        pltpu.VMEM((CHUNK, EMB), jnp.float32),
        pltpu.VMEM((CHUNK, EMB), jnp.float32),
        pltpu.SemaphoreType.DMA,
        pltpu.SemaphoreType.DMA,
        pltpu.SemaphoreType.DMA,
        pltpu.SemaphoreType.DMA,
    ],
)
def _emb_lookup(idx_hbm, table_hbm, out_hbm, mask_hbm,
                idx_v, mask_v, buf0, buf1, g0, g1, w0, w1):
    wid = lax.axis_index("s") * 2 + lax.axis_index("c")
    base = wid * PER_W

    pltpu.sync_copy(idx_hbm.at[pl.ds(base, PER_W)], idx_v)

    def body(j, carry):
        pltpu.async_copy(
            table_hbm.at[idx_v.at[pl.ds(j * CHUNK, CHUNK)]], buf0, g0
        ).wait()
        pltpu.sync_copy(buf0, out_hbm.at[pl.ds(base + j * CHUNK, CHUNK)])
        return carry

    lax.fori_loop(0, NCH, body, 0)

    def mask_body(i, carry):
        s = i * 16
        v = idx_v[pl.ds(s, 16)]
        one = jnp.full((16,), 1, jnp.int32)
        neg = jnp.full((16,), -1, jnp.int32)
        zero = jnp.full((16,), 0, jnp.int32)
        mask_v[pl.ds(s, 16)] = jnp.where(v > 0, one, jnp.where(v < 0, neg, zero))
        return carry

    lax.fori_loop(0, PER_W // 16, mask_body, 0)
    pltpu.sync_copy(mask_v, mask_hbm.at[pl.ds(base, PER_W)])


def kernel(context, table):
    ctx_flat = context.reshape(TOT)
    emb, mask = _emb_lookup(ctx_flat, table)
    return emb.reshape(B_ROWS, SEQ, EMB), mask.reshape(B_ROWS, SEQ)


# speedup vs baseline: 3.3195x; 1.1306x over previous
import functools

import jax
import jax.numpy as jnp
from jax import lax
from jax.experimental import pallas as pl
from jax.experimental.pallas import tpu as pltpu
from jax.experimental.pallas import tpu_sc as plsc

EMB = 128
B_ROWS = 4096
SEQ = 50
TOT = B_ROWS * SEQ
NW = 32
PER_W = TOT // NW
CHUNK = 128
NCH = PER_W // CHUNK
NBUF = 4
NGRP = NCH // NBUF

_mesh = plsc.VectorSubcoreMesh(core_axis_name="c", subcore_axis_name="s")


def _sign16(v):
    one = jnp.full((16,), 1, jnp.int32)
    neg = jnp.full((16,), -1, jnp.int32)
    zero = jnp.full((16,), 0, jnp.int32)
    return jnp.where(v > 0, one, jnp.where(v < 0, neg, zero))


@functools.partial(
    pl.kernel,
    mesh=_mesh,
    out_type=(
        jax.ShapeDtypeStruct((TOT, EMB), jnp.float32),
        jax.ShapeDtypeStruct((TOT,), jnp.int32),
    ),
    scratch_types=[
        pltpu.VMEM((PER_W,), jnp.int32),
        pltpu.VMEM((PER_W,), jnp.int32),
        pltpu.VMEM((CHUNK, EMB), jnp.float32),
        pltpu.VMEM((CHUNK, EMB), jnp.float32),
        pltpu.VMEM((CHUNK, EMB), jnp.float32),
        pltpu.VMEM((CHUNK, EMB), jnp.float32),
        pltpu.SemaphoreType.DMA,
        pltpu.SemaphoreType.DMA,
        pltpu.SemaphoreType.DMA,
        pltpu.SemaphoreType.DMA,
        pltpu.SemaphoreType.DMA,
        pltpu.SemaphoreType.DMA,
        pltpu.SemaphoreType.DMA,
        pltpu.SemaphoreType.DMA,
    ],
)
def _emb_lookup(idx_hbm, table_hbm, out_hbm, mask_hbm,
                idx_v, mask_v, buf0, buf1, buf2, buf3,
                g0, g1, g2, g3, w0, w1, w2, w3):
    wid = lax.axis_index("s") * 2 + lax.axis_index("c")
    base = wid * PER_W

    bufs = (buf0, buf1, buf2, buf3)
    gs = (g0, g1, g2, g3)
    ws = (w0, w1, w2, w3)

    pltpu.sync_copy(idx_hbm.at[pl.ds(base, PER_W)], idx_v)

    def gather(j, b):
        pltpu.async_copy(
            table_hbm.at[idx_v.at[pl.ds(j * CHUNK, CHUNK)]], bufs[b], gs[b]
        )

    def gather_wait(j, b):
        pltpu.make_async_copy(
            table_hbm.at[idx_v.at[pl.ds(j * CHUNK, CHUNK)]], bufs[b], gs[b]
        ).wait()

    def write(j, b):
        pltpu.async_copy(
            bufs[b], out_hbm.at[pl.ds(base + j * CHUNK, CHUNK)], ws[b]
        )

    def write_wait(j, b):
        pltpu.make_async_copy(
            bufs[b], out_hbm.at[pl.ds(base + j * CHUNK, CHUNK)], ws[b]
        ).wait()

    def mask_chunk(j):
        for k in range(CHUNK // 16):
            s = j * CHUNK + k * 16
            mask_v[pl.ds(s, 16)] = _sign16(idx_v[pl.ds(s, 16)])

    gather(0, 0)
    gather(1, 1)

    def group(jj, carry):
        j0 = jj * NBUF
        for b in range(NBUF):
            j = j0 + b
            b2 = (b + 2) % NBUF
            gather_wait(j, b)
            write(j, b)
            mask_chunk(j)

            @pl.when(j >= 2)
            def _():
                write_wait(j - 2, b2)

            @pl.when(j + 2 < NCH)
            def _():
                gather(j + 2, b2)

        return carry

    lax.fori_loop(0, NGRP, group, 0)

    for b, j in ((0, NCH - 2), (1, NCH - 1)):
        gather_wait(j, b)
        write(j, b)
        mask_chunk(j)

    pltpu.sync_copy(mask_v, mask_hbm.at[pl.ds(base, PER_W)])

    for j in range(NCH - 4, NCH):
        write_wait(j, j % NBUF)


def kernel(context, table):
    ctx_flat = context.reshape(TOT)
    emb, mask = _emb_lookup(ctx_flat, table)
    return emb.reshape(B_ROWS, SEQ, EMB), mask.reshape(B_ROWS, SEQ)


# speedup vs baseline: 5.4455x; 1.6404x over previous
import functools

import jax
import jax.numpy as jnp
from jax import lax
from jax.experimental import pallas as pl
from jax.experimental.pallas import tpu as pltpu
from jax.experimental.pallas import tpu_sc as plsc

EMB = 128
B_ROWS = 4096
SEQ = 50
NW = 32
ROWS_W = B_ROWS // NW
NBUF = 4
NGRP = ROWS_W // NBUF

_mesh = plsc.VectorSubcoreMesh(core_axis_name="c", subcore_axis_name="s")


@functools.partial(
    pl.kernel,
    mesh=_mesh,
    out_type=jax.ShapeDtypeStruct((B_ROWS, SEQ, EMB), jnp.float32),
    scratch_types=[
        pltpu.VMEM((ROWS_W, SEQ), jnp.int32),
        pltpu.VMEM((SEQ, EMB), jnp.float32),
        pltpu.VMEM((SEQ, EMB), jnp.float32),
        pltpu.VMEM((SEQ, EMB), jnp.float32),
        pltpu.VMEM((SEQ, EMB), jnp.float32),
        pltpu.SemaphoreType.DMA,
        pltpu.SemaphoreType.DMA,
        pltpu.SemaphoreType.DMA,
        pltpu.SemaphoreType.DMA,
        pltpu.SemaphoreType.DMA,
        pltpu.SemaphoreType.DMA,
        pltpu.SemaphoreType.DMA,
        pltpu.SemaphoreType.DMA,
    ],
)
def _emb_lookup(ctx_hbm, table_hbm, out_hbm,
                idx_v, buf0, buf1, buf2, buf3,
                g0, g1, g2, g3, w0, w1, w2, w3):
    wid = lax.axis_index("s") * 2 + lax.axis_index("c")
    r0 = wid * ROWS_W

    bufs = (buf0, buf1, buf2, buf3)
    gs = (g0, g1, g2, g3)
    ws = (w0, w1, w2, w3)

    pltpu.sync_copy(ctx_hbm.at[pl.ds(r0, ROWS_W)], idx_v)

    def gather(j, b):
        pltpu.async_copy(table_hbm.at[idx_v.at[j]], bufs[b], gs[b])

    def gather_wait(j, b):
        pltpu.make_async_copy(
            table_hbm.at[idx_v.at[j]], bufs[b], gs[b]
        ).wait()

    def write(j, b):
        pltpu.async_copy(bufs[b], out_hbm.at[r0 + j], ws[b])

    def write_wait(j, b):
        pltpu.make_async_copy(bufs[b], out_hbm.at[r0 + j], ws[b]).wait()

    gather(0, 0)
    gather(1, 1)

    def group(jj, carry):
        j0 = jj * NBUF
        for b in range(NBUF):
            j = j0 + b
            b2 = (b + 2) % NBUF
            gather_wait(j, b)
            write(j, b)

            @pl.when(j >= 2)
            def _():
                write_wait(j - 2, b2)

            @pl.when(j + 2 < ROWS_W)
            def _():
                gather(j + 2, b2)

        return carry

    lax.fori_loop(0, NGRP, group, 0)

    for j in (ROWS_W - 2, ROWS_W - 1):
        write_wait(j, j % NBUF)


def _mask_body(ctx_ref, out_ref):
    out_ref[...] = jnp.sign(ctx_ref[...])


_mask = pl.pallas_call(
    _mask_body,
    out_shape=jax.ShapeDtypeStruct((B_ROWS, SEQ), jnp.int32),
)


def kernel(context, table):
    emb = _emb_lookup(context, table)
    return emb, _mask(context)


# speedup vs baseline: 5.4494x; 1.0007x over previous
import functools

import jax
import jax.numpy as jnp
from jax import lax
from jax.experimental import pallas as pl
from jax.experimental.pallas import tpu as pltpu
from jax.experimental.pallas import tpu_sc as plsc

EMB = 128
B_ROWS = 4096
SEQ = 50
NW = 32
ROWS_W = B_ROWS // NW
NBUF = 4
NGRP = ROWS_W // NBUF

_mesh = plsc.VectorSubcoreMesh(core_axis_name="c", subcore_axis_name="s")


@functools.partial(
    pl.kernel,
    mesh=_mesh,
    compiler_params=pltpu.CompilerParams(use_tc_tiling_on_sc=True),
    out_type=jax.ShapeDtypeStruct((B_ROWS, SEQ, EMB), jnp.float32),
    scratch_types=[
        pltpu.VMEM((ROWS_W, SEQ), jnp.int32),
        pltpu.VMEM((SEQ, EMB), jnp.float32),
        pltpu.VMEM((SEQ, EMB), jnp.float32),
        pltpu.VMEM((SEQ, EMB), jnp.float32),
        pltpu.VMEM((SEQ, EMB), jnp.float32),
        pltpu.SemaphoreType.DMA,
        pltpu.SemaphoreType.DMA,
        pltpu.SemaphoreType.DMA,
        pltpu.SemaphoreType.DMA,
        pltpu.SemaphoreType.DMA,
        pltpu.SemaphoreType.DMA,
        pltpu.SemaphoreType.DMA,
        pltpu.SemaphoreType.DMA,
    ],
)
def _emb_lookup(ctx_hbm, table_hbm, out_hbm,
                idx_v, buf0, buf1, buf2, buf3,
                g0, g1, g2, g3, w0, w1, w2, w3):
    wid = lax.axis_index("s") * 2 + lax.axis_index("c")
    r0 = wid * ROWS_W

    bufs = (buf0, buf1, buf2, buf3)
    gs = (g0, g1, g2, g3)
    ws = (w0, w1, w2, w3)

    pltpu.sync_copy(ctx_hbm.at[pl.ds(r0, ROWS_W)], idx_v)

    def gather(j, b):
        pltpu.async_copy(table_hbm.at[idx_v.at[j]], bufs[b], gs[b])

    def gather_wait(j, b):
        pltpu.make_async_copy(
            table_hbm.at[idx_v.at[j]], bufs[b], gs[b]
        ).wait()

    def write(j, b):
        pltpu.async_copy(bufs[b], out_hbm.at[r0 + j], ws[b])

    def write_wait(j, b):
        pltpu.make_async_copy(bufs[b], out_hbm.at[r0 + j], ws[b]).wait()

    gather(0, 0)
    gather(1, 1)

    def group(jj, carry):
        j0 = jj * NBUF
        for b in range(NBUF):
            j = j0 + b
            b2 = (b + 2) % NBUF
            gather_wait(j, b)
            write(j, b)

            @pl.when(j >= 2)
            def _():
                write_wait(j - 2, b2)

            @pl.when(j + 2 < ROWS_W)
            def _():
                gather(j + 2, b2)

        return carry

    lax.fori_loop(0, NGRP, group, 0)

    for j in (ROWS_W - 2, ROWS_W - 1):
        write_wait(j, j % NBUF)


def _mask_body(ctx_ref, out_ref):
    out_ref[...] = jnp.sign(ctx_ref[...])


_mask = pl.pallas_call(
    _mask_body,
    out_shape=jax.ShapeDtypeStruct((B_ROWS, SEQ), jnp.int32),
)


def kernel(context, table):
    emb = _emb_lookup(context, table)
    return emb, _mask(context)


# speedup vs baseline: 5.9004x; 1.0828x over previous
import functools

import jax
import jax.numpy as jnp
from jax import lax
from jax.experimental import pallas as pl
from jax.experimental.pallas import tpu as pltpu
from jax.experimental.pallas import tpu_sc as plsc

EMB = 128
B_ROWS = 4096
SEQ = 50
NW = 32
ROWS_W = B_ROWS // NW
NBUF = 8
DEPTH = 6
NGRP = ROWS_W // NBUF

_mesh = plsc.VectorSubcoreMesh(core_axis_name="c", subcore_axis_name="s")


@functools.partial(
    pl.kernel,
    mesh=_mesh,
    compiler_params=pltpu.CompilerParams(use_tc_tiling_on_sc=True),
    out_type=jax.ShapeDtypeStruct((B_ROWS, SEQ, EMB), jnp.float32),
    scratch_types=[
        pltpu.VMEM((ROWS_W, SEQ), jnp.int32),
    ] + [pltpu.VMEM((SEQ, EMB), jnp.float32)] * NBUF
      + [pltpu.SemaphoreType.DMA] * (2 * NBUF),
)
def _emb_lookup(ctx_hbm, table_hbm, out_hbm, idx_v, *bs):
    wid = lax.axis_index("s") * 2 + lax.axis_index("c")
    r0 = wid * ROWS_W

    bufs = bs[:NBUF]
    gs = bs[NBUF:2 * NBUF]
    ws = bs[2 * NBUF:]

    pltpu.sync_copy(ctx_hbm.at[pl.ds(r0, ROWS_W)], idx_v)

    def gather(j, b):
        pltpu.async_copy(table_hbm.at[idx_v.at[j]], bufs[b], gs[b])

    def gather_wait(j, b):
        pltpu.make_async_copy(
            table_hbm.at[idx_v.at[j]], bufs[b], gs[b]
        ).wait()

    def write(j, b):
        pltpu.async_copy(bufs[b], out_hbm.at[r0 + j], ws[b])

    def write_wait(j, b):
        pltpu.make_async_copy(bufs[b], out_hbm.at[r0 + j], ws[b]).wait()

    for j in range(DEPTH):
        gather(j, j)

    def group(jj, carry):
        j0 = jj * NBUF
        for b in range(NBUF):
            j = j0 + b
            br = (b + DEPTH) % NBUF
            gather_wait(j, b)
            write(j, b)

            @pl.when(j + DEPTH >= NBUF)
            def _():
                write_wait(j + DEPTH - NBUF, br)

            @pl.when(j + DEPTH < ROWS_W)
            def _():
                gather(j + DEPTH, br)

        return carry

    lax.fori_loop(0, NGRP, group, 0)

    for j in range(ROWS_W - (NBUF - DEPTH), ROWS_W):
        write_wait(j, j % NBUF)


def _mask_body(ctx_ref, out_ref):
    out_ref[...] = jnp.sign(ctx_ref[...])


_mask = pl.pallas_call(
    _mask_body,
    out_shape=jax.ShapeDtypeStruct((B_ROWS, SEQ), jnp.int32),
)


def kernel(context, table):
    emb = _emb_lookup(context, table)
    return emb, _mask(context)
